# baseline (speedup 1.0000x reference)
import dataclasses
import functools

import jax
import jax.numpy as jnp
from jax import lax
from jax.experimental import pallas as pl
from jax.experimental.pallas import tpu as pltpu
from jax.experimental.pallas import tpu_sc as plsc

_P = 0.02
_NC = 2
_NS = 16
_NW = _NC * _NS
_CHUNK = 6400


def _fused_sc(table, idx_flat, t_flat, seq):
    n = idx_flat.shape[0]
    per_w = n // _NW
    rows_w = per_w // seq
    win_rows = _CHUNK // seq
    assert win_rows * seq == _CHUNK
    mesh = plsc.VectorSubcoreMesh(core_axis_name="c", subcore_axis_name="s")

    v = table.shape[0]
    stage = 5000
    n_slices = v // stage
    assert n_slices * stage == v
    n_rounds = -(-n_slices // _NS)

    n_win = per_w // _CHUNK
    assert n_win % 2 == 0 and n_win * _CHUNK == per_w

    inv_s = 1.0 / seq

    cp = pltpu.CompilerParams()
    if "needs_layout_passes" in pltpu.CompilerParams.__dataclass_fields__:
        cp = dataclasses.replace(cp, needs_layout_passes=False)

    @functools.partial(
        pl.kernel,
        out_type=jax.ShapeDtypeStruct((n,), jnp.float32),
        mesh=mesh,
        compiler_params=cp,
        scratch_types=[
            pltpu.VMEM((_CHUNK,), jnp.int32),
            pltpu.VMEM((_CHUNK,), jnp.int32),
            pltpu.VMEM((_CHUNK,), jnp.float32),
            pltpu.VMEM((_CHUNK,), jnp.float32),
            pltpu.VMEM((rows_w,), jnp.float32),
            pltpu.VMEM_SHARED((v,), jnp.float32),
            pltpu.SemaphoreType.DMA,
            pltpu.SemaphoreType.DMA,
            pltpu.SemaphoreType.DMA,
            pltpu.SemaphoreType.DMA,
            pltpu.SemaphoreType.DMA,
            pltpu.SemaphoreType.DMA,
        ],
    )
    def fused_kernel(table_hbm, idx_hbm, t_hbm, out_hbm, ib0, ib1, vb0, vb1,
                     t_all, table_sp, ia0, ia1, oa0, oa1, g0, g1):
        sid = lax.axis_index("s")
        wid = sid * _NC + lax.axis_index("c")
        base = wid * per_w
        row0 = wid * rows_w

        pltpu.async_copy(t_hbm.at[pl.ds(row0, rows_w)], t_all, g0)

        for r in range(n_rounds):
            slice_id = r * _NS + sid

            @pl.when(slice_id < n_slices)
            def _():
                so = slice_id * stage
                pltpu.sync_copy(table_hbm.at[pl.ds(so, stage)],
                                vb0.at[pl.ds(0, stage)])
                pltpu.sync_copy(vb0.at[pl.ds(0, stage)],
                                table_sp.at[pl.ds(so, stage)])

        pltpu.make_async_copy(t_hbm.at[pl.ds(0, rows_w)], t_all, g0).wait()
        plsc.subcore_barrier()

        lanes = lax.iota(jnp.int32, 16)
        m0f = jnp.where(lanes < 8, 1.0, 0.0)
        m1f = 1.0 - m0f
        nv = (2 * seq) // 16

        def finish_window(vb, woff):
            @pl.loop(0, win_rows, step=16)
            def _(g):
                tg = t_all[pl.ds(woff * win_rows + g, 16)]
                for j in range(8):
                    rp = 2 * j
                    pbase = (g + rp) * seq
                    t0 = tg[rp]
                    t1 = tg[rp + 1]
                    xs = [vb[pl.ds(pbase + 16 * i, 16)] for i in range(nv)]
                    acc0 = xs[0]
                    for i in range(1, 12):
                        acc0 = acc0 + xs[i]
                    acc1 = xs[13]
                    for i in range(14, nv):
                        acc1 = acc1 + xs[i]
                    s0v = jnp.full((16,), jnp.sum(acc0 + xs[12] * m0f))
                    s1v = jnp.full((16,), jnp.sum(acc1 + xs[12] * m1f))
                    t0v = jnp.full((16,), t0)
                    t1v = jnp.full((16,), t1)
                    mu0 = s0v * inv_s
                    mu1 = s1v * inv_s
                    fd0 = t0v / (mu0 + 1e-10)
                    fd1 = t1v / (mu1 + 1e-10)
                    fu0 = (1.0 - t0v) / (1.0 - mu0 + 1e-10)
                    fu1 = (1.0 - t1v) / (1.0 - mu1 + 1e-10)
                    sel0 = mu0 > t0v
                    sel1 = mu1 > t1v
                    a0 = jnp.where(sel0, fd0, fu0)
                    b0 = jnp.where(sel0, 0.0, 1.0 - fu0)
                    a1 = jnp.where(sel1, fd1, fu1)
                    b1 = jnp.where(sel1, 0.0, 1.0 - fu1)
                    av = a0 * m0f + a1 * m1f
                    bv = b0 * m0f + b1 * m1f
                    for i in range(nv):
                        if i < 12:
                            r = a0 * xs[i] + b0
                        elif i == 12:
                            r = av * xs[i] + bv
                        else:
                            r = a1 * xs[i] + b1
                        r = jnp.minimum(jnp.maximum(r, 0.0), 1.0)
                        vb[pl.ds(pbase + 16 * i, 16)] = r

        pltpu.async_copy(idx_hbm.at[pl.ds(base, _CHUNK)], ib0, ia0)
        pltpu.async_copy(idx_hbm.at[pl.ds(base + _CHUNK, _CHUNK)], ib1, ia1)
        pltpu.make_async_copy(idx_hbm.at[pl.ds(0, _CHUNK)], ib0, ia0).wait()
        pltpu.async_copy(table_sp.at[ib0], vb0, g0)

        @pl.loop(0, n_win, step=2)
        def _(w):
            for k, (ib, vb, ia, oa, g, ibn, vbn, ian, oan, gn) in enumerate((
                    (ib0, vb0, ia0, oa0, g0, ib1, vb1, ia1, oa1, g1),
                    (ib1, vb1, ia1, oa1, g1, ib0, vb0, ia0, oa0, g0))):
                off = base + (w + k) * _CHUNK

                @pl.when(w + k + 1 < n_win)
                def _():
                    pltpu.make_async_copy(
                        idx_hbm.at[pl.ds(0, _CHUNK)], ibn, ian).wait()

                    @pl.when(w + k >= 2)
                    def _():
                        pltpu.make_async_copy(
                            out_hbm.at[pl.ds(0, _CHUNK)], vbn, oan).wait()

                    pltpu.async_copy(table_sp.at[ibn], vbn, gn)

                pltpu.make_async_copy(
                    table_hbm.at[pl.ds(0, _CHUNK)], vb, g).wait()

                @pl.when(w + k + 2 < n_win)
                def _():
                    pltpu.async_copy(
                        idx_hbm.at[pl.ds(off + 2 * _CHUNK, _CHUNK)], ib, ia)

                finish_window(vb, w + k)
                pltpu.async_copy(vb, out_hbm.at[pl.ds(off, _CHUNK)], oa)

        pltpu.make_async_copy(out_hbm.at[pl.ds(0, _CHUNK)], vb0, oa0).wait()
        pltpu.make_async_copy(out_hbm.at[pl.ds(0, _CHUNK)], vb1, oa1).wait()

    return fused_kernel(table, idx_flat, t_flat)


def _soften_tc(table):
    v = table.shape[0]

    def body(w_ref, o_ref):
        o_ref[...] = jnp.exp(_P * jnp.log(w_ref[...]))

    return pl.pallas_call(
        body,
        out_shape=jax.ShapeDtypeStruct((v,), jnp.float32),
    )(table)


def _squeeze_t_tc(t_col):
    b = t_col.shape[0]

    def body(t_ref, o_ref):
        o_ref[...] = t_ref[...].reshape(-1)

    return pl.pallas_call(
        body,
        out_shape=jax.ShapeDtypeStruct((b,), jnp.float32),
    )(t_col)


def kernel(base_weights, x, target_mask_rate):
    b, s = x.shape
    softened_table = _soften_tc(base_weights)
    out_flat = _fused_sc(softened_table, x.reshape(-1),
                         target_mask_rate.reshape(-1), s)
    return out_flat.reshape(b, s)


# speedup vs baseline: 1.0136x; 1.0136x over previous
import dataclasses
import functools

import jax
import jax.numpy as jnp
from jax import lax
from jax.experimental import pallas as pl
from jax.experimental.pallas import tpu as pltpu
from jax.experimental.pallas import tpu_sc as plsc

_P = 0.02
_NC = 2
_NS = 16
_NW = _NC * _NS
_CHUNK = 12800


def _fused_sc(table, idx_flat, t_flat, seq):
    n = idx_flat.shape[0]
    per_w = n // _NW
    rows_w = per_w // seq
    win_rows = _CHUNK // seq
    assert win_rows * seq == _CHUNK
    mesh = plsc.VectorSubcoreMesh(core_axis_name="c", subcore_axis_name="s")

    v = table.shape[0]
    stage = 10000
    n_slices = v // stage
    assert n_slices * stage == v
    n_rounds = -(-n_slices // _NS)

    n_win = per_w // _CHUNK
    assert n_win % 2 == 0 and n_win * _CHUNK == per_w

    inv_s = 1.0 / seq

    cp = pltpu.CompilerParams()
    if "needs_layout_passes" in pltpu.CompilerParams.__dataclass_fields__:
        cp = dataclasses.replace(cp, needs_layout_passes=False)

    @functools.partial(
        pl.kernel,
        out_type=jax.ShapeDtypeStruct((n,), jnp.float32),
        mesh=mesh,
        compiler_params=cp,
        scratch_types=[
            pltpu.VMEM((_CHUNK,), jnp.int32),
            pltpu.VMEM((_CHUNK,), jnp.int32),
            pltpu.VMEM((_CHUNK,), jnp.float32),
            pltpu.VMEM((_CHUNK,), jnp.float32),
            pltpu.VMEM((rows_w,), jnp.float32),
            pltpu.VMEM_SHARED((v,), jnp.float32),
            pltpu.SemaphoreType.DMA,
            pltpu.SemaphoreType.DMA,
            pltpu.SemaphoreType.DMA,
            pltpu.SemaphoreType.DMA,
            pltpu.SemaphoreType.DMA,
            pltpu.SemaphoreType.DMA,
        ],
    )
    def fused_kernel(table_hbm, idx_hbm, t_hbm, out_hbm, ib0, ib1, vb0, vb1,
                     t_all, table_sp, ia0, ia1, oa0, oa1, g0, g1):
        sid = lax.axis_index("s")
        wid = sid * _NC + lax.axis_index("c")
        base = wid * per_w
        row0 = wid * rows_w

        pltpu.async_copy(t_hbm.at[pl.ds(row0, rows_w)], t_all, g0)

        for r in range(n_rounds):
            slice_id = r * _NS + sid

            @pl.when(slice_id < n_slices)
            def _():
                so = slice_id * stage
                pltpu.sync_copy(table_hbm.at[pl.ds(so, stage)],
                                vb0.at[pl.ds(0, stage)])
                pltpu.sync_copy(vb0.at[pl.ds(0, stage)],
                                table_sp.at[pl.ds(so, stage)])

        pltpu.make_async_copy(t_hbm.at[pl.ds(0, rows_w)], t_all, g0).wait()
        plsc.subcore_barrier()

        lanes = lax.iota(jnp.int32, 16)
        m0f = jnp.where(lanes < 8, 1.0, 0.0)
        m1f = 1.0 - m0f
        nv = (2 * seq) // 16

        def finish_window(vb, woff):
            @pl.loop(0, win_rows, step=16)
            def _(g):
                tg = t_all[pl.ds(woff * win_rows + g, 16)]
                for j in range(8):
                    rp = 2 * j
                    pbase = (g + rp) * seq
                    t0 = tg[rp]
                    t1 = tg[rp + 1]
                    xs = [vb[pl.ds(pbase + 16 * i, 16)] for i in range(nv)]
                    acc0 = xs[0]
                    for i in range(1, 12):
                        acc0 = acc0 + xs[i]
                    acc1 = xs[13]
                    for i in range(14, nv):
                        acc1 = acc1 + xs[i]
                    s0v = jnp.full((16,), jnp.sum(acc0 + xs[12] * m0f))
                    s1v = jnp.full((16,), jnp.sum(acc1 + xs[12] * m1f))
                    t0v = jnp.full((16,), t0)
                    t1v = jnp.full((16,), t1)
                    mu0 = s0v * inv_s
                    mu1 = s1v * inv_s
                    fd0 = t0v / (mu0 + 1e-10)
                    fd1 = t1v / (mu1 + 1e-10)
                    fu0 = (1.0 - t0v) / (1.0 - mu0 + 1e-10)
                    fu1 = (1.0 - t1v) / (1.0 - mu1 + 1e-10)
                    sel0 = mu0 > t0v
                    sel1 = mu1 > t1v
                    a0 = jnp.where(sel0, fd0, fu0)
                    b0 = jnp.where(sel0, 0.0, 1.0 - fu0)
                    a1 = jnp.where(sel1, fd1, fu1)
                    b1 = jnp.where(sel1, 0.0, 1.0 - fu1)
                    av = a0 * m0f + a1 * m1f
                    bv = b0 * m0f + b1 * m1f
                    for i in range(nv):
                        if i < 12:
                            r = a0 * xs[i] + b0
                        elif i == 12:
                            r = av * xs[i] + bv
                        else:
                            r = a1 * xs[i] + b1
                        r = jnp.minimum(jnp.maximum(r, 0.0), 1.0)
                        vb[pl.ds(pbase + 16 * i, 16)] = r

        pltpu.async_copy(idx_hbm.at[pl.ds(base, _CHUNK)], ib0, ia0)
        pltpu.async_copy(idx_hbm.at[pl.ds(base + _CHUNK, _CHUNK)], ib1, ia1)
        pltpu.make_async_copy(idx_hbm.at[pl.ds(0, _CHUNK)], ib0, ia0).wait()
        pltpu.async_copy(table_sp.at[ib0], vb0, g0)

        @pl.loop(0, n_win, step=2)
        def _(w):
            for k, (ib, vb, ia, oa, g, ibn, vbn, ian, oan, gn) in enumerate((
                    (ib0, vb0, ia0, oa0, g0, ib1, vb1, ia1, oa1, g1),
                    (ib1, vb1, ia1, oa1, g1, ib0, vb0, ia0, oa0, g0))):
                off = base + (w + k) * _CHUNK

                @pl.when(w + k + 1 < n_win)
                def _():
                    pltpu.make_async_copy(
                        idx_hbm.at[pl.ds(0, _CHUNK)], ibn, ian).wait()

                    @pl.when(w + k >= 2)
                    def _():
                        pltpu.make_async_copy(
                            out_hbm.at[pl.ds(0, _CHUNK)], vbn, oan).wait()

                    pltpu.async_copy(table_sp.at[ibn], vbn, gn)

                pltpu.make_async_copy(
                    table_hbm.at[pl.ds(0, _CHUNK)], vb, g).wait()

                @pl.when(w + k + 2 < n_win)
                def _():
                    pltpu.async_copy(
                        idx_hbm.at[pl.ds(off + 2 * _CHUNK, _CHUNK)], ib, ia)

                finish_window(vb, w + k)
                pltpu.async_copy(vb, out_hbm.at[pl.ds(off, _CHUNK)], oa)

        pltpu.make_async_copy(out_hbm.at[pl.ds(0, _CHUNK)], vb0, oa0).wait()
        pltpu.make_async_copy(out_hbm.at[pl.ds(0, _CHUNK)], vb1, oa1).wait()

    return fused_kernel(table, idx_flat, t_flat)


def _soften_tc(table):
    v = table.shape[0]

    def body(w_ref, o_ref):
        o_ref[...] = jnp.exp(_P * jnp.log(w_ref[...]))

    return pl.pallas_call(
        body,
        out_shape=jax.ShapeDtypeStruct((v,), jnp.float32),
    )(table)


def _squeeze_t_tc(t_col):
    b = t_col.shape[0]

    def body(t_ref, o_ref):
        o_ref[...] = t_ref[...].reshape(-1)

    return pl.pallas_call(
        body,
        out_shape=jax.ShapeDtypeStruct((b,), jnp.float32),
    )(t_col)


def kernel(base_weights, x, target_mask_rate):
    b, s = x.shape
    softened_table = _soften_tc(base_weights)
    out_flat = _fused_sc(softened_table, x.reshape(-1),
                         target_mask_rate.reshape(-1), s)
    return out_flat.reshape(b, s)
